# baseline (speedup 1.0000x reference)
import jax
import jax.numpy as jnp
from jax import lax
from jax.experimental import pallas as pl
from jax.experimental.pallas import tpu as pltpu

_NPS = 1024
_N = 4096
_B = 2
_BM = 512


def _dist_kernel(f_ref, ft_ref, o_ref):
    o_ref[0] = lax.dot_general(
        f_ref[0], ft_ref[0], (((1,), (0,)), ((), ())),
        preferred_element_type=jnp.float32)


def _fps_kernel(dist_ref, asq_ref, asqc_ref, P_ref, PT_ref, out_ref,
                row0_s, row1_s, sem0, sem1):
    rows = (row0_s, row1_s)
    sems = (sem0, sem1)

    pos = (lax.broadcasted_iota(jnp.int32, (8, 128), 0) * 128
           + lax.broadcasted_iota(jnp.int32, (8, 128), 1))

    def row_copy(b, l):
        return pltpu.make_async_copy(
            dist_ref.at[b, pl.ds(l, 1), :], rows[b], sems[b])

    def argmax_flat(md):
        return jnp.argmax(md, axis=1)[0].astype(jnp.int32)

    def argmax_first(md):
        return jnp.argmax(md.reshape(1, _N), axis=1)[0].astype(jnp.int32)

    for b in range(_B):
        row_copy(b, 0).start()

    init_mf = jnp.full((1, _N), 1e10, dtype=jnp.float32)
    init_md = jnp.full((8, 512), 1e10, dtype=jnp.float32)
    zeros_acc = jnp.zeros((8, 128), jnp.int32)
    carry0 = (init_mf, init_mf, init_md, init_md,
              jnp.int32(0), jnp.int32(0), jnp.int32(0), jnp.int32(0),
              zeros_acc, zeros_acc, zeros_acc, zeros_acc)

    def body(t, c):
        mf = [c[0], c[1]]
        mdp = [c[2], c[3]]
        lf = [c[4], c[5]]
        ldp = [c[6], c[7]]
        af = [c[8], c[9]]
        adp = [c[10], c[11]]
        for b in range(_B):
            l = lf[b]
            row_copy(b, l).wait()
            a_l = asqc_ref[b, pl.ds(l, 1), :][0, 0]
            b_row = asq_ref[b:b + 1, :]
            row = (-2.0 * rows[b][...] + a_l) + b_row
            mf[b] = jnp.minimum(mf[b], row)
        for b in range(_B):
            l = ldp[b]
            px = PT_ref[b, 0]
            py = PT_ref[b, 1]
            pz = PT_ref[b, 2]
            cen = P_ref[b, pl.ds(l, 1), :]
            c0 = cen[0, 0]
            c1 = cen[0, 1]
            c2 = cen[0, 2]
            dx = px - c0
            dy = py - c1
            dz = pz - c2
            row = (dx * dx + dy * dy) + dz * dz
            mdp[b] = jnp.minimum(mdp[b], row)
        nf = [argmax_flat(mf[b]) for b in range(_B)]
        nd = [argmax_first(mdp[b]) for b in range(_B)]
        for b in range(_B):
            row_copy(b, nf[b]).start()
            lf[b] = nf[b]
            af[b] = jnp.where(pos == t, nf[b], af[b])
            ldp[b] = nd[b]
            adp[b] = jnp.where(pos == t, nd[b], adp[b])
        return (mf[0], mf[1], mdp[0], mdp[1],
                lf[0], lf[1], ldp[0], ldp[1],
                af[0], af[1], adp[0], adp[1])

    cN = lax.fori_loop(1, _NPS, body, carry0)
    for b in range(_B):
        row_copy(b, cN[4 + b]).wait()
    for b in range(_B):
        out_ref[0, b] = cN[8 + b]
        out_ref[1, b] = cN[10 + b]


def kernel(points, features, npoint):
    F = jnp.concatenate([points, jnp.swapaxes(features, 1, 2)], axis=2)
    asq = jnp.sum(F ** 2, axis=-1)
    FT = jnp.swapaxes(F, 1, 2)
    PT8 = jnp.swapaxes(points, 1, 2).reshape(2, 3, 8, 512)

    dist = pl.pallas_call(
        _dist_kernel,
        grid=(_B, _N // _BM),
        in_specs=[
            pl.BlockSpec((1, _BM, 131), lambda b, i: (b, i, 0)),
            pl.BlockSpec((1, 131, _N), lambda b, i: (b, 0, 0)),
        ],
        out_specs=pl.BlockSpec((1, _BM, _N), lambda b, i: (b, i, 0)),
        out_shape=jax.ShapeDtypeStruct((_B, _N, _N), jnp.float32),
    )(F, FT)

    out = pl.pallas_call(
        _fps_kernel,
        in_specs=[
            pl.BlockSpec(memory_space=pl.ANY),
            pl.BlockSpec(memory_space=pltpu.MemorySpace.VMEM),
            pl.BlockSpec(memory_space=pltpu.MemorySpace.VMEM),
            pl.BlockSpec(memory_space=pltpu.MemorySpace.VMEM),
            pl.BlockSpec(memory_space=pltpu.MemorySpace.VMEM),
        ],
        scratch_shapes=[
            pltpu.VMEM((1, _N), jnp.float32),
            pltpu.VMEM((1, _N), jnp.float32),
            pltpu.SemaphoreType.DMA,
            pltpu.SemaphoreType.DMA,
        ],
        out_shape=jax.ShapeDtypeStruct((2, _B, 8, 128), jnp.int32),
    )(dist, asq, asq[..., None], points, PT8)

    idx = out.reshape(2, _B, _NPS)
    fps_idx = jnp.concatenate([idx[0], idx[1]], axis=1)
    return fps_idx + (jnp.asarray(npoint, dtype=jnp.int32) - _NPS)


# speedup vs baseline: 1.0207x; 1.0207x over previous
import jax
import jax.numpy as jnp
from jax import lax
from jax.experimental import pallas as pl
from jax.experimental.pallas import tpu as pltpu

_NPS = 1024
_N = 4096
_B = 2
_BM = 512


def _dist_kernel(f_ref, ft_ref, o_ref):
    o_ref[0] = lax.dot_general(
        f_ref[0], ft_ref[0], (((1,), (0,)), ((), ())),
        preferred_element_type=jnp.float32)


def _fps_kernel(dist_ref, asq_ref, asqc_ref, P_ref, PT_ref, out_ref,
                row0_s, row1_s, sem0, sem1):
    rows = (row0_s, row1_s)
    sems = (sem0, sem1)

    pos = (lax.broadcasted_iota(jnp.int32, (8, 128), 0) * 128
           + lax.broadcasted_iota(jnp.int32, (8, 128), 1))

    def row_copy(b, l):
        return pltpu.make_async_copy(
            dist_ref.at[b, pl.ds(l, 1), :], rows[b], sems[b])

    def argmax_flat(md):
        return jnp.argmax(md, axis=1)[0].astype(jnp.int32)

    def argmax_first(md):
        return jnp.argmax(md.reshape(1, _N), axis=1)[0].astype(jnp.int32)

    init_md = jnp.full((8, 512), 1e10, dtype=jnp.float32)
    zeros_acc = jnp.zeros((8, 128), jnp.int32)

    def fold_row(b, md, l):
        a_l = asqc_ref[b, pl.ds(l, 1), :][0, 0]
        b_row = asq_ref[b:b + 1, :]
        row = (-2.0 * rows[b][...] + a_l) + b_row
        return jnp.minimum(md, row)

    mf0 = []
    for b in range(_B):
        row_copy(b, 0).start()
    for b in range(_B):
        row_copy(b, 0).wait()
        mf0.append(fold_row(b, jnp.full((1, _N), 1e10, jnp.float32), 0))

    carry0 = (mf0[0], mf0[1], init_md, init_md,
              jnp.int32(0), jnp.int32(0),
              zeros_acc, zeros_acc, zeros_acc, zeros_acc)

    def body(t, c):
        mf = [c[0], c[1]]
        mdp = [c[2], c[3]]
        ldp = [c[4], c[5]]
        af = [c[6], c[7]]
        adp = [c[8], c[9]]
        nf = [argmax_flat(mf[b]) for b in range(_B)]
        for b in range(_B):
            row_copy(b, nf[b]).start()
            af[b] = jnp.where(pos == t, nf[b], af[b])
        for b in range(_B):
            l = ldp[b]
            px = PT_ref[b, 0]
            py = PT_ref[b, 1]
            pz = PT_ref[b, 2]
            cen = P_ref[b, pl.ds(l, 1), :]
            c0 = cen[0, 0]
            c1 = cen[0, 1]
            c2 = cen[0, 2]
            dx = px - c0
            dy = py - c1
            dz = pz - c2
            row = (dx * dx + dy * dy) + dz * dz
            md = jnp.minimum(mdp[b], row)
            nd = argmax_first(md)
            mdp[b] = md
            ldp[b] = nd
            adp[b] = jnp.where(pos == t, nd, adp[b])
        for b in range(_B):
            row_copy(b, nf[b]).wait()
            mf[b] = fold_row(b, mf[b], nf[b])
        return (mf[0], mf[1], mdp[0], mdp[1],
                ldp[0], ldp[1],
                af[0], af[1], adp[0], adp[1])

    cN = lax.fori_loop(1, _NPS, body, carry0)
    for b in range(_B):
        out_ref[0, b] = cN[6 + b]
        out_ref[1, b] = cN[8 + b]


def kernel(points, features, npoint):
    F = jnp.concatenate([points, jnp.swapaxes(features, 1, 2)], axis=2)
    asq = jnp.sum(F ** 2, axis=-1)
    FT = jnp.swapaxes(F, 1, 2)
    PT8 = jnp.swapaxes(points, 1, 2).reshape(2, 3, 8, 512)

    dist = pl.pallas_call(
        _dist_kernel,
        grid=(_B, _N // _BM),
        in_specs=[
            pl.BlockSpec((1, _BM, 131), lambda b, i: (b, i, 0)),
            pl.BlockSpec((1, 131, _N), lambda b, i: (b, 0, 0)),
        ],
        out_specs=pl.BlockSpec((1, _BM, _N), lambda b, i: (b, i, 0)),
        out_shape=jax.ShapeDtypeStruct((_B, _N, _N), jnp.float32),
    )(F, FT)

    out = pl.pallas_call(
        _fps_kernel,
        in_specs=[
            pl.BlockSpec(memory_space=pl.ANY),
            pl.BlockSpec(memory_space=pltpu.MemorySpace.VMEM),
            pl.BlockSpec(memory_space=pltpu.MemorySpace.VMEM),
            pl.BlockSpec(memory_space=pltpu.MemorySpace.VMEM),
            pl.BlockSpec(memory_space=pltpu.MemorySpace.VMEM),
        ],
        scratch_shapes=[
            pltpu.VMEM((1, _N), jnp.float32),
            pltpu.VMEM((1, _N), jnp.float32),
            pltpu.SemaphoreType.DMA,
            pltpu.SemaphoreType.DMA,
        ],
        out_shape=jax.ShapeDtypeStruct((2, _B, 8, 128), jnp.int32),
    )(dist, asq, asq[..., None], points, PT8)

    idx = out.reshape(2, _B, _NPS)
    fps_idx = jnp.concatenate([idx[0], idx[1]], axis=1)
    return fps_idx + (jnp.asarray(npoint, dtype=jnp.int32) - _NPS)


# speedup vs baseline: 1.5202x; 1.4894x over previous
import jax
import jax.numpy as jnp
from jax import lax
from jax.experimental import pallas as pl
from jax.experimental.pallas import tpu as pltpu

_NPS = 1024
_N = 4096
_B = 2


def _fps_kernel(F_ref, FT_ref, asq_ref, asqc_ref, P_ref, PT_ref, out_ref):
    pos = (lax.broadcasted_iota(jnp.int32, (8, 128), 0) * 128
           + lax.broadcasted_iota(jnp.int32, (8, 128), 1))

    def argmax_flat(md):
        return jnp.argmax(md, axis=1)[0].astype(jnp.int32)

    def argmax_first(md):
        return jnp.argmax(md.reshape(1, _N), axis=1)[0].astype(jnp.int32)

    def fold_row(b, md, l):
        fr = F_ref[b, pl.ds(l, 1), :]
        mv = lax.dot_general(
            fr, FT_ref[b], (((1,), (0,)), ((), ())),
            preferred_element_type=jnp.float32)
        a_l = asqc_ref[b, pl.ds(l, 1), :][0, 0]
        b_row = asq_ref[b:b + 1, :]
        row = (-2.0 * mv + a_l) + b_row
        return jnp.minimum(md, row)

    init_md = jnp.full((8, 512), 1e10, dtype=jnp.float32)
    zeros_acc = jnp.zeros((8, 128), jnp.int32)

    mf0 = [fold_row(b, jnp.full((1, _N), 1e10, jnp.float32), 0)
           for b in range(_B)]

    carry0 = (mf0[0], mf0[1], init_md, init_md,
              jnp.int32(0), jnp.int32(0),
              zeros_acc, zeros_acc, zeros_acc, zeros_acc)

    def body(t, c):
        mf = [c[0], c[1]]
        mdp = [c[2], c[3]]
        ldp = [c[4], c[5]]
        af = [c[6], c[7]]
        adp = [c[8], c[9]]
        nf = [argmax_flat(mf[b]) for b in range(_B)]
        for b in range(_B):
            af[b] = jnp.where(pos == t, nf[b], af[b])
        for b in range(_B):
            l = ldp[b]
            px = PT_ref[b, 0]
            py = PT_ref[b, 1]
            pz = PT_ref[b, 2]
            cen = P_ref[b, pl.ds(l, 1), :]
            c0 = cen[0, 0]
            c1 = cen[0, 1]
            c2 = cen[0, 2]
            dx = px - c0
            dy = py - c1
            dz = pz - c2
            row = (dx * dx + dy * dy) + dz * dz
            md = jnp.minimum(mdp[b], row)
            nd = argmax_first(md)
            mdp[b] = md
            ldp[b] = nd
            adp[b] = jnp.where(pos == t, nd, adp[b])
        for b in range(_B):
            mf[b] = fold_row(b, mf[b], nf[b])
        return (mf[0], mf[1], mdp[0], mdp[1],
                ldp[0], ldp[1],
                af[0], af[1], adp[0], adp[1])

    cN = lax.fori_loop(1, _NPS, body, carry0)
    for b in range(_B):
        out_ref[0, b] = cN[6 + b]
        out_ref[1, b] = cN[8 + b]


def kernel(points, features, npoint):
    F = jnp.concatenate([points, jnp.swapaxes(features, 1, 2)], axis=2)
    asq = jnp.sum(F ** 2, axis=-1)
    FT = jnp.swapaxes(F, 1, 2)
    PT8 = jnp.swapaxes(points, 1, 2).reshape(2, 3, 8, 512)

    out = pl.pallas_call(
        _fps_kernel,
        out_shape=jax.ShapeDtypeStruct((2, _B, 8, 128), jnp.int32),
    )(F, FT, asq, asq[..., None], points, PT8)

    idx = out.reshape(2, _B, _NPS)
    fps_idx = jnp.concatenate([idx[0], idx[1]], axis=1)
    return fps_idx + (jnp.asarray(npoint, dtype=jnp.int32) - _NPS)
